# baseline (speedup 1.0000x reference)
import functools

import jax
import jax.numpy as jnp
from jax import lax
from jax.experimental import pallas as pl
from jax.experimental.pallas import tpu as pltpu
from jax.experimental.pallas import tpu_sc as plsc

N_EXPERTS = 64
TOP_K = 8
TOK_BLOCK_TC = 1024


def _tc_logits_body(x_ref, w_ref, out_ref):
    out_ref[...] = lax.dot_general(
        x_ref[...],
        w_ref[...],
        dimension_numbers=(((1,), (1,)), ((), ())),
        preferred_element_type=jnp.float32,
    )


def _tc_logits(x, w):
    t, h = x.shape
    return pl.pallas_call(
        _tc_logits_body,
        grid=(t // TOK_BLOCK_TC,),
        in_specs=[
            pl.BlockSpec((TOK_BLOCK_TC, h), lambda i: (i, 0)),
            pl.BlockSpec((N_EXPERTS, h), lambda i: (0, 0)),
        ],
        out_specs=pl.BlockSpec((TOK_BLOCK_TC, N_EXPERTS), lambda i: (i, 0)),
        out_shape=jax.ShapeDtypeStruct((t, N_EXPERTS), jnp.float32),
    )(x, w)


def _sc_topk(logits):
    t = logits.shape[0] // N_EXPERTS
    info = plsc.get_sparse_core_info()
    nc, ns, lanes = info.num_cores, info.num_subcores, info.num_lanes
    nw = nc * ns
    per_w = t // nw
    n_blocks = per_w // lanes
    mesh = plsc.VectorSubcoreMesh(core_axis_name="c", subcore_axis_name="s")

    @functools.partial(
        pl.kernel,
        mesh=mesh,
        out_type=[
            jax.ShapeDtypeStruct((t * TOP_K,), jnp.float32),
            jax.ShapeDtypeStruct((t * TOP_K,), jnp.int32),
        ],
        scratch_types=[
            pltpu.VMEM((lanes * N_EXPERTS,), jnp.float32),
            pltpu.VMEM((per_w * TOP_K,), jnp.float32),
            pltpu.VMEM((per_w * TOP_K,), jnp.int32),
        ],
        compiler_params=pltpu.CompilerParams(needs_layout_passes=False),
    )
    def k(logits_hbm, outw_hbm, outi_hbm, lblk, wv, iv):
        wid = lax.axis_index("s") * nc + lax.axis_index("c")
        base = wid * per_w
        rows = lax.iota(jnp.int32, lanes)
        rows_scaled = rows * N_EXPERTS
        neg = jnp.full((lanes,), -jnp.inf, jnp.float32)

        def block(b, carry):
            tok0 = base + b * lanes
            pltpu.sync_copy(
                logits_hbm.at[pl.ds(tok0 * N_EXPERTS, lanes * N_EXPERTS)], lblk
            )
            tvals = [neg] * TOP_K
            tidx = [jnp.zeros((lanes,), jnp.int32)] * TOP_K
            for e in range(N_EXPERTS):
                x = plsc.load_gather(lblk, [rows_scaled + e])
                xi = jnp.full((lanes,), e, jnp.int32)
                for j in range(TOP_K):
                    c = x > tvals[j]
                    hi = jnp.maximum(tvals[j], x)
                    lo = jnp.minimum(tvals[j], x)
                    ii = jnp.where(c, xi, tidx[j])
                    xi = jnp.where(c, tidx[j], xi)
                    tvals[j] = hi
                    x = lo
                    tidx[j] = ii
            exps = [jnp.exp(tvals[j] - tvals[0]) for j in range(TOP_K)]
            s = exps[0]
            for j in range(1, TOP_K):
                s = s + exps[j]
            r = 1.0 / s
            loc = (b * lanes + rows) * TOP_K
            for j in range(TOP_K):
                plsc.store_scatter(wv, [loc + j], exps[j] * r)
                plsc.store_scatter(iv, [loc + j], tidx[j])
            return carry

        lax.fori_loop(0, n_blocks, block, 0)
        pltpu.sync_copy(wv, outw_hbm.at[pl.ds(base * TOP_K, per_w * TOP_K)])
        pltpu.sync_copy(iv, outi_hbm.at[pl.ds(base * TOP_K, per_w * TOP_K)])

    return k(logits)


N_CHUNKS = 4


def kernel(hidden_states, weight):
    b, s, h = hidden_states.shape
    x = hidden_states.reshape(-1, h)
    t = x.shape[0]
    tc = t // N_CHUNKS
    ws, idxs = [], []
    for c in range(N_CHUNKS):
        logits = _tc_logits(x[c * tc : (c + 1) * tc], weight)
        w_flat, i_flat = _sc_topk(logits.reshape(-1))
        ws.append(w_flat.reshape(tc, TOP_K))
        idxs.append(i_flat.reshape(tc, TOP_K))
    return jnp.concatenate(ws, axis=0), jnp.concatenate(idxs, axis=0)


# speedup vs baseline: 1.7181x; 1.7181x over previous
import functools

import jax
import jax.numpy as jnp
from jax import lax
from jax.experimental import pallas as pl
from jax.experimental.pallas import tpu as pltpu
from jax.experimental.pallas import tpu_sc as plsc

N_EXPERTS = 64
TOP_K = 8
TOK_BLOCK_TC = 1024


def _tc_logits_body(x_ref, w_ref, out_ref):
    out_ref[...] = lax.dot_general(
        x_ref[...],
        w_ref[...],
        dimension_numbers=(((1,), (0,)), ((), ())),
        preferred_element_type=jnp.float32,
    )


def _tc_logits(x, wt):
    t, h = x.shape
    return pl.pallas_call(
        _tc_logits_body,
        grid=(t // TOK_BLOCK_TC,),
        in_specs=[
            pl.BlockSpec((TOK_BLOCK_TC, h), lambda i: (i, 0)),
            pl.BlockSpec((h, N_EXPERTS), lambda i: (0, 0)),
        ],
        out_specs=pl.BlockSpec((TOK_BLOCK_TC, N_EXPERTS), lambda i: (i, 0)),
        out_shape=jax.ShapeDtypeStruct((t, N_EXPERTS), jnp.float32),
    )(x, wt)


def _sc_topk(logits):
    t = logits.shape[0] // N_EXPERTS
    info = plsc.get_sparse_core_info()
    nc, ns, lanes = info.num_cores, info.num_subcores, info.num_lanes
    nw = nc * ns
    per_w = t // nw
    n_blocks = per_w // lanes
    blk = lanes * N_EXPERTS
    mesh = plsc.VectorSubcoreMesh(core_axis_name="c", subcore_axis_name="s")

    @functools.partial(
        pl.kernel,
        mesh=mesh,
        out_type=[
            jax.ShapeDtypeStruct((t * TOP_K,), jnp.float32),
            jax.ShapeDtypeStruct((t * TOP_K,), jnp.int32),
        ],
        scratch_types=[
            pltpu.VMEM((blk,), jnp.float32),
            pltpu.VMEM((blk,), jnp.float32),
            pltpu.VMEM((per_w * TOP_K,), jnp.float32),
            pltpu.VMEM((per_w * TOP_K,), jnp.int32),
            pltpu.SemaphoreType.DMA,
            pltpu.SemaphoreType.DMA,
        ],
        compiler_params=pltpu.CompilerParams(needs_layout_passes=False),
    )
    def k(logits_hbm, outw_hbm, outi_hbm, lblk0, lblk1, wv, iv, sem0, sem1):
        wid = lax.axis_index("s") * nc + lax.axis_index("c")
        base = wid * per_w
        rows = lax.iota(jnp.int32, lanes)
        rows_scaled = rows * N_EXPERTS
        neg = jnp.full((lanes,), -jnp.inf, jnp.float32)
        sems = (sem0, sem1)
        bufs = (lblk0, lblk1)

        def src(b):
            return logits_hbm.at[pl.ds((base + b * lanes) * N_EXPERTS, blk)]

        def compute(buf, b):
            tvals = [neg] * TOP_K
            tidx = [jnp.zeros((lanes,), jnp.int32)] * TOP_K
            for e in range(N_EXPERTS):
                x = plsc.load_gather(buf, [rows_scaled + e])
                xi = jnp.full((lanes,), e, jnp.int32)
                for j in range(TOP_K):
                    c = x > tvals[j]
                    hi = jnp.maximum(tvals[j], x)
                    lo = jnp.minimum(tvals[j], x)
                    ii = jnp.where(c, xi, tidx[j])
                    xi = jnp.where(c, tidx[j], xi)
                    tvals[j] = hi
                    x = lo
                    tidx[j] = ii
            exps = [jnp.exp(tvals[j] - tvals[0]) for j in range(TOP_K)]
            s = exps[0]
            for j in range(1, TOP_K):
                s = s + exps[j]
            r = 1.0 / s
            loc = (b * lanes + rows) * TOP_K
            for j in range(TOP_K):
                plsc.store_scatter(wv, [loc + j], exps[j] * r)
                plsc.store_scatter(iv, [loc + j], tidx[j])

        pltpu.async_copy(src(0), lblk0, sem0)
        pltpu.async_copy(src(1), lblk1, sem1)

        def pair(g, carry):
            b0 = 2 * g
            for q in range(2):
                b = b0 + q
                buf, sem = bufs[q], sems[q]
                pltpu.make_async_copy(src(b), buf, sem).wait()
                compute(buf, b)
                nxt = jnp.minimum(b + 2, n_blocks - 1)
                pltpu.async_copy(src(nxt), buf, sem)
            return carry

        lax.fori_loop(0, n_blocks // 2, pair, 0)
        pltpu.make_async_copy(src(0), lblk0, sem0).wait()
        pltpu.make_async_copy(src(1), lblk1, sem1).wait()
        pltpu.sync_copy(wv, outw_hbm.at[pl.ds(base * TOP_K, per_w * TOP_K)])
        pltpu.sync_copy(iv, outi_hbm.at[pl.ds(base * TOP_K, per_w * TOP_K)])

    return k(logits)


def kernel(hidden_states, weight):
    b, s, h = hidden_states.shape
    x = hidden_states.reshape(-1, h)
    t = x.shape[0]
    logits = _tc_logits(x, weight.T)
    w_flat, i_flat = _sc_topk(logits.reshape(-1))
    return w_flat.reshape(t, TOP_K), i_flat.reshape(t, TOP_K)


# speedup vs baseline: 1.9199x; 1.1175x over previous
import functools

import jax
import jax.numpy as jnp
from jax import lax
from jax.experimental import pallas as pl
from jax.experimental.pallas import tpu as pltpu
from jax.experimental.pallas import tpu_sc as plsc

N_EXPERTS = 64
TOP_K = 8
TOK_BLOCK_TC = 512


def _tc_keys_body(x_ref, w_ref, out_ref):
    logits = lax.dot_general(
        x_ref[...],
        w_ref[...],
        dimension_numbers=(((1,), (0,)), ((), ())),
        preferred_element_type=jnp.float32,
    )
    y = lax.bitcast_convert_type(logits, jnp.int32)
    si = y ^ (lax.shift_right_arithmetic(y, 31) & jnp.int32(0x7FFFFFFF))
    e = lax.broadcasted_iota(jnp.int32, logits.shape, 1)
    out_ref[...] = (si & jnp.int32(-64)) | (jnp.int32(N_EXPERTS - 1) - e)


def _tc_keys(x, wt):
    t, h = x.shape
    return pl.pallas_call(
        _tc_keys_body,
        grid=(t // TOK_BLOCK_TC,),
        in_specs=[
            pl.BlockSpec((TOK_BLOCK_TC, h), lambda i: (i, 0)),
            pl.BlockSpec((h, N_EXPERTS), lambda i: (0, 0)),
        ],
        out_specs=pl.BlockSpec((TOK_BLOCK_TC, N_EXPERTS), lambda i: (i, 0)),
        out_shape=jax.ShapeDtypeStruct((t, N_EXPERTS), jnp.int32),
    )(x, wt)


def _sc_topk(keys):
    t = keys.shape[0] // N_EXPERTS
    info = plsc.get_sparse_core_info()
    nc, ns, lanes = info.num_cores, info.num_subcores, info.num_lanes
    nw = nc * ns
    per_w = t // nw
    n_blocks = per_w // lanes
    blk = lanes * N_EXPERTS
    mesh = plsc.VectorSubcoreMesh(core_axis_name="c", subcore_axis_name="s")

    @functools.partial(
        pl.kernel,
        mesh=mesh,
        out_type=[
            jax.ShapeDtypeStruct((t * TOP_K,), jnp.float32),
            jax.ShapeDtypeStruct((t * TOP_K,), jnp.int32),
        ],
        scratch_types=[
            pltpu.VMEM((blk,), jnp.int32),
            pltpu.VMEM((blk,), jnp.int32),
            pltpu.VMEM((per_w * TOP_K,), jnp.float32),
            pltpu.VMEM((per_w * TOP_K,), jnp.int32),
            pltpu.SemaphoreType.DMA,
            pltpu.SemaphoreType.DMA,
        ],
        compiler_params=pltpu.CompilerParams(needs_layout_passes=False),
    )
    def k(keys_hbm, outw_hbm, outi_hbm, lblk0, lblk1, wv, iv, sem0, sem1):
        wid = lax.axis_index("s") * nc + lax.axis_index("c")
        base = wid * per_w
        rows = lax.iota(jnp.int32, lanes)
        rows_scaled = rows * N_EXPERTS
        bot = jnp.full((lanes,), jnp.iinfo(jnp.int32).min, jnp.int32)
        sems = (sem0, sem1)
        bufs = (lblk0, lblk1)

        def src(b):
            return keys_hbm.at[pl.ds((base + b * lanes) * N_EXPERTS, blk)]

        def compute(buf, b):
            tkey = [bot] * TOP_K
            for e in range(N_EXPERTS):
                x = plsc.load_gather(buf, [rows_scaled + e])
                for j in range(TOP_K):
                    hi = jnp.maximum(tkey[j], x)
                    x = jnp.minimum(tkey[j], x)
                    tkey[j] = hi
            vals, idxs = [], []
            for j in range(TOP_K):
                idxs.append(jnp.int32(N_EXPERTS - 1) - (tkey[j] & jnp.int32(63)))
                sk = tkey[j] & jnp.int32(-64)
                y = sk ^ (lax.shift_right_arithmetic(sk, 31) & jnp.int32(0x7FFFFFFF))
                vals.append(lax.bitcast_convert_type(y, jnp.float32))
            exps = [jnp.exp(vals[j] - vals[0]) for j in range(TOP_K)]
            s = exps[0]
            for j in range(1, TOP_K):
                s = s + exps[j]
            r = 1.0 / s
            loc = (b * lanes + rows) * TOP_K
            for j in range(TOP_K):
                plsc.store_scatter(wv, [loc + j], exps[j] * r)
                plsc.store_scatter(iv, [loc + j], idxs[j])

        pltpu.async_copy(src(0), lblk0, sem0)
        pltpu.async_copy(src(1), lblk1, sem1)

        def pair(g, carry):
            b0 = 2 * g
            for q in range(2):
                b = b0 + q
                buf, sem = bufs[q], sems[q]
                pltpu.make_async_copy(src(b), buf, sem).wait()
                compute(buf, b)
                nxt = jnp.minimum(b + 2, n_blocks - 1)
                pltpu.async_copy(src(nxt), buf, sem)
            return carry

        lax.fori_loop(0, n_blocks // 2, pair, 0)
        pltpu.make_async_copy(src(0), lblk0, sem0).wait()
        pltpu.make_async_copy(src(1), lblk1, sem1).wait()
        pltpu.sync_copy(wv, outw_hbm.at[pl.ds(base * TOP_K, per_w * TOP_K)])
        pltpu.sync_copy(iv, outi_hbm.at[pl.ds(base * TOP_K, per_w * TOP_K)])

    return k(keys)


def kernel(hidden_states, weight):
    b, s, h = hidden_states.shape
    x = hidden_states.reshape(-1, h)
    t = x.shape[0]
    keys = _tc_keys(x, weight.T)
    w_flat, i_flat = _sc_topk(keys.reshape(-1))
    return w_flat.reshape(t, TOP_K), i_flat.reshape(t, TOP_K)


# speedup vs baseline: 1.9934x; 1.0383x over previous
import functools

import jax
import jax.numpy as jnp
from jax import lax
from jax.experimental import pallas as pl
from jax.experimental.pallas import tpu as pltpu
from jax.experimental.pallas import tpu_sc as plsc

N_EXPERTS = 64
TOP_K = 8
TOK_BLOCK_TC = 512
KEY_SCALE = float(1 << 19)


def _tc_keys_body(x_ref, w_ref, out_ref):
    logits = lax.dot_general(
        x_ref[...],
        w_ref[...],
        dimension_numbers=(((1,), (0,)), ((), ())),
        preferred_element_type=jnp.float32,
    )
    fix = jnp.clip(
        logits * jnp.float32(KEY_SCALE), -33554000.0, 33554000.0
    ).astype(jnp.int32)
    e = lax.broadcasted_iota(jnp.int32, logits.shape, 1)
    out_ref[...] = lax.shift_left(fix, 6) | (jnp.int32(N_EXPERTS - 1) - e)


def _tc_keys(x, wt):
    t, h = x.shape
    return pl.pallas_call(
        _tc_keys_body,
        grid=(t // TOK_BLOCK_TC,),
        in_specs=[
            pl.BlockSpec((TOK_BLOCK_TC, h), lambda i: (i, 0)),
            pl.BlockSpec((h, N_EXPERTS), lambda i: (0, 0)),
        ],
        out_specs=pl.BlockSpec((TOK_BLOCK_TC, N_EXPERTS), lambda i: (i, 0)),
        out_shape=jax.ShapeDtypeStruct((t, N_EXPERTS), jnp.int32),
    )(x, wt)


def _sc_topk(keys):
    t = keys.shape[0] // N_EXPERTS
    info = plsc.get_sparse_core_info()
    nc, ns, lanes = info.num_cores, info.num_subcores, info.num_lanes
    nw = nc * ns
    per_w = t // nw
    n_blocks = per_w // lanes
    blk = lanes * N_EXPERTS
    mesh = plsc.VectorSubcoreMesh(core_axis_name="c", subcore_axis_name="s")

    @functools.partial(
        pl.kernel,
        mesh=mesh,
        out_type=[
            jax.ShapeDtypeStruct((t * TOP_K,), jnp.float32),
            jax.ShapeDtypeStruct((t * TOP_K,), jnp.int32),
        ],
        scratch_types=[
            pltpu.VMEM((blk,), jnp.int32),
            pltpu.VMEM((blk,), jnp.int32),
            pltpu.VMEM((per_w * TOP_K,), jnp.float32),
            pltpu.VMEM((per_w * TOP_K,), jnp.int32),
            pltpu.SemaphoreType.DMA,
            pltpu.SemaphoreType.DMA,
        ],
        compiler_params=pltpu.CompilerParams(needs_layout_passes=False),
    )
    def k(keys_hbm, outw_hbm, outi_hbm, lblk0, lblk1, wv, iv, sem0, sem1):
        wid = lax.axis_index("s") * nc + lax.axis_index("c")
        base = wid * per_w
        rows = lax.iota(jnp.int32, lanes)
        rows_scaled = rows * N_EXPERTS
        bot = jnp.full((lanes,), jnp.iinfo(jnp.int32).min, jnp.int32)
        sems = (sem0, sem1)
        bufs = (lblk0, lblk1)

        def src(b):
            return keys_hbm.at[pl.ds((base + b * lanes) * N_EXPERTS, blk)]

        sort8_pairs = (
            (0, 1), (2, 3), (4, 5), (6, 7),
            (0, 2), (1, 3), (4, 6), (5, 7),
            (1, 2), (5, 6),
            (0, 4), (1, 5), (2, 6), (3, 7),
            (2, 4), (3, 5),
            (1, 2), (3, 4), (5, 6),
        )

        def sort8(g):
            for i, j in sort8_pairs:
                hi = jnp.maximum(g[i], g[j])
                g[j] = jnp.minimum(g[i], g[j])
                g[i] = hi
            return g

        def compute(buf, b):
            tkey = sort8([
                plsc.load_gather(buf, [rows_scaled + e]) for e in range(TOP_K)
            ])
            for g0 in range(TOP_K, N_EXPERTS, TOP_K):
                g = sort8([
                    plsc.load_gather(buf, [rows_scaled + (g0 + q)])
                    for q in range(TOP_K)
                ])
                h = [jnp.maximum(tkey[i], g[TOP_K - 1 - i]) for i in range(TOP_K)]
                for d in (4, 2, 1):
                    for i in range(TOP_K):
                        if (i // d) % 2 == 0:
                            hi = jnp.maximum(h[i], h[i + d])
                            h[i + d] = jnp.minimum(h[i], h[i + d])
                            h[i] = hi
                tkey = h
            idxs = [
                jnp.int32(N_EXPERTS - 1) - (tkey[j] & jnp.int32(63))
                for j in range(TOP_K)
            ]
            fixs = [lax.shift_right_arithmetic(tkey[j], 6) for j in range(TOP_K)]
            exps = [
                jnp.exp(
                    (fixs[j] - fixs[0]).astype(jnp.float32)
                    * jnp.float32(1.0 / KEY_SCALE)
                )
                for j in range(TOP_K)
            ]
            s = exps[0]
            for j in range(1, TOP_K):
                s = s + exps[j]
            r = 1.0 / s
            loc = (b * lanes + rows) * TOP_K
            for j in range(TOP_K):
                plsc.store_scatter(wv, [loc + j], exps[j] * r)
                plsc.store_scatter(iv, [loc + j], idxs[j])

        pltpu.async_copy(src(0), lblk0, sem0)
        pltpu.async_copy(src(1), lblk1, sem1)

        def pair(g, carry):
            b0 = 2 * g
            for q in range(2):
                b = b0 + q
                buf, sem = bufs[q], sems[q]
                pltpu.make_async_copy(src(b), buf, sem).wait()
                compute(buf, b)
                nxt = jnp.minimum(b + 2, n_blocks - 1)
                pltpu.async_copy(src(nxt), buf, sem)
            return carry

        lax.fori_loop(0, n_blocks // 2, pair, 0)
        pltpu.make_async_copy(src(0), lblk0, sem0).wait()
        pltpu.make_async_copy(src(1), lblk1, sem1).wait()
        pltpu.sync_copy(wv, outw_hbm.at[pl.ds(base * TOP_K, per_w * TOP_K)])
        pltpu.sync_copy(iv, outi_hbm.at[pl.ds(base * TOP_K, per_w * TOP_K)])

    return k(keys)


def kernel(hidden_states, weight):
    b, s, h = hidden_states.shape
    x = hidden_states.reshape(-1, h)
    t = x.shape[0]
    keys = _tc_keys(x, weight.T)
    w_flat, i_flat = _sc_topk(keys.reshape(-1))
    return w_flat.reshape(t, TOP_K), i_flat.reshape(t, TOP_K)
